# baseline (speedup 1.0000x reference)
import jax
import jax.numpy as jnp
from jax.experimental import pallas as pl
from jax.experimental.pallas import tpu as pltpu

_D = 4096
_NR = 1024
_R_OUT = 64
_KT = 512
_BM = 256
_NK = _D // _KT
_HB = 4


def _fused_kernel(wids_ref, x_ref, al_ref, ws_ref, out_ref, w2l_ref, acc_ref):
    h = pl.program_id(0)
    k = pl.program_id(1)
    i = pl.program_id(2)

    xb = x_ref[...].astype(jnp.bfloat16)

    @pl.when((h == 0) & (i == 0))
    def _build_large():
        for e in range(16):
            w2l_ref[pl.ds(k * _KT, _KT), e * 64:(e + 1) * 64] = (
                al_ref[e].astype(jnp.bfloat16))

    @pl.when(h == 0)
    def _dot_large():
        part = jnp.dot(xb, w2l_ref[pl.ds(k * _KT, _KT), :],
                       preferred_element_type=jnp.float32)
        @pl.when(k == 0)
        def _init():
            acc_ref[i] = part
        @pl.when(k > 0)
        def _accum():
            acc_ref[i] += part

    @pl.when(h == 1)
    def _dot_small():
        part = jnp.dot(xb, ws_ref[...], preferred_element_type=jnp.float32)
        @pl.when(k == 0)
        def _init():
            acc_ref[i] = part
        @pl.when(k > 0)
        def _accum():
            acc_ref[i] += part

    @pl.when(k == _NK - 1)
    def _finish():
        shift = jnp.where(h == 0, 6, 4)
        rmask = jnp.where(h == 0, 63, 15)
        acc = acc_ref[i]
        wid = wids_ref[0, 0, :]
        lane_e = jax.lax.broadcasted_iota(jnp.int32, (_BM, _NR), 1) >> shift
        masked = jnp.where(wid[:, None] == lane_e, acc, 0.0).astype(jnp.bfloat16)
        gi = jax.lax.broadcasted_iota(jnp.int32, (_NR, _R_OUT), 0)
        gj = jax.lax.broadcasted_iota(jnp.int32, (_NR, _R_OUT), 1)
        fold = ((gi & rmask) == gj).astype(jnp.bfloat16)
        out_ref[...] = jnp.dot(masked, fold, preferred_element_type=jnp.float32)


def kernel(x, wids_large, wids_small, lora_A_large, lora_A_small):
    b_l = wids_large.shape[0]
    b_s = wids_small.shape[0]
    n_l, d, r_l = lora_A_large.shape
    n_s, _, r_s = lora_A_small.shape
    nblk = (b_l + b_s) // _BM

    wids3 = jnp.concatenate([wids_large, wids_small]).reshape(nblk, 1, _BM)
    ws2 = lora_A_small.transpose(1, 0, 2).reshape(d, n_s * r_s).astype(jnp.bfloat16)

    out = pl.pallas_call(
        _fused_kernel,
        grid=(2, _NK, _HB),
        in_specs=[
            pl.BlockSpec((1, 1, _BM), lambda h, k, i: (h * _HB + i, 0, 0)),
            pl.BlockSpec((_BM, _KT), lambda h, k, i: (h * _HB + i, k)),
            pl.BlockSpec((n_l, _KT, r_l),
                         lambda h, k, i: (0, jnp.where(h == 0, k, _NK - 1), 0)),
            pl.BlockSpec((_KT, _NR),
                         lambda h, k, i: (jnp.where(h == 1, k, 0), 0)),
        ],
        out_specs=pl.BlockSpec((_BM, _R_OUT), lambda h, k, i: (h * _HB + i, 0)),
        out_shape=jax.ShapeDtypeStruct((b_l + b_s, _R_OUT), jnp.float32),
        scratch_shapes=[
            pltpu.VMEM((_D, _NR), jnp.bfloat16),
            pltpu.VMEM((_HB, _BM, _NR), jnp.float32),
        ],
    )(wids3, x.reshape(b_l + b_s, d), lora_A_large, ws2)

    yl = out[:b_l, :r_l]
    ys = out[b_l:, :r_s]
    return (yl[:, None, :], ys[:, None, :])


# speedup vs baseline: 1.0606x; 1.0606x over previous
import jax
import jax.numpy as jnp
from jax.experimental import pallas as pl
from jax.experimental.pallas import tpu as pltpu

_D = 4096
_NR = 1024
_R_OUT = 64
_KT = 512
_BM = 256
_NK = _D // _KT
_HB = 4
_NL = 16


def _fused_kernel(*refs):
    wids_ref = refs[0]
    x_ref = refs[1]
    a_refs = refs[2:2 + _NL]
    ws_ref = refs[2 + _NL]
    out_ref = refs[3 + _NL]
    w2_ref = refs[4 + _NL]
    acc_ref = refs[5 + _NL]

    h = pl.program_id(0)
    i = pl.program_id(1)
    k = pl.program_id(2)
    ks = pl.ds(k * _KT, _KT)

    @pl.when((h == 0) & (i == 0))
    def _build_large():
        for e in range(_NL):
            w2_ref[ks, e * 64:(e + 1) * 64] = a_refs[e][...].astype(jnp.bfloat16)

    @pl.when((h == 1) & (i == 0))
    def _build_small():
        w2_ref[ks, :] = ws_ref[...]

    xb = x_ref[...].astype(jnp.bfloat16)
    part = jnp.dot(xb, w2_ref[ks, :], preferred_element_type=jnp.float32)

    @pl.when(k == 0)
    def _init():
        acc_ref[...] = part

    @pl.when(k > 0)
    def _accum():
        acc_ref[...] += part

    @pl.when(k == _NK - 1)
    def _finish():
        shift = jnp.where(h == 0, 6, 4)
        rmask = jnp.where(h == 0, 63, 15)
        acc = acc_ref[...]
        wid = wids_ref[0, 0, :]
        lane_e = jax.lax.broadcasted_iota(jnp.int32, (_BM, _NR), 1) >> shift
        masked = jnp.where(wid[:, None] == lane_e, acc, 0.0).astype(jnp.bfloat16)
        gi = jax.lax.broadcasted_iota(jnp.int32, (_NR, _R_OUT), 0)
        gj = jax.lax.broadcasted_iota(jnp.int32, (_NR, _R_OUT), 1)
        fold = ((gi & rmask) == gj).astype(jnp.bfloat16)
        out_ref[...] = jnp.dot(masked, fold, preferred_element_type=jnp.float32)


def _idx_large(e):
    def idx(h, i, k, e=e):
        return (e * _NK + jnp.where((h == 0) & (i == 0), k, _NK - 1), 0)
    return idx


def kernel(x, wids_large, wids_small, lora_A_large, lora_A_small):
    b_l = wids_large.shape[0]
    b_s = wids_small.shape[0]
    n_l, d, r_l = lora_A_large.shape
    n_s, _, r_s = lora_A_small.shape
    nblk = (b_l + b_s) // _BM

    wids3 = jnp.concatenate([wids_large, wids_small]).reshape(nblk, 1, _BM)
    al2 = lora_A_large.reshape(n_l * d, r_l)
    ws2 = lora_A_small.transpose(1, 0, 2).reshape(d, n_s * r_s).astype(jnp.bfloat16)

    in_specs = [
        pl.BlockSpec((1, 1, _BM), lambda h, i, k: (h * _HB + i, 0, 0)),
        pl.BlockSpec((_BM, _KT), lambda h, i, k: (h * _HB + i, k)),
    ]
    in_specs += [pl.BlockSpec((_KT, r_l), _idx_large(e)) for e in range(n_l)]
    in_specs += [
        pl.BlockSpec((_KT, _NR),
                     lambda h, i, k: (jnp.where((h == 1) & (i == 0), k, _NK - 1), 0)),
    ]

    out = pl.pallas_call(
        _fused_kernel,
        grid=(2, _HB, _NK),
        in_specs=in_specs,
        out_specs=pl.BlockSpec((_BM, _R_OUT), lambda h, i, k: (h * _HB + i, 0)),
        out_shape=jax.ShapeDtypeStruct((b_l + b_s, _R_OUT), jnp.float32),
        scratch_shapes=[
            pltpu.VMEM((_D, _NR), jnp.bfloat16),
            pltpu.VMEM((_BM, _NR), jnp.float32),
        ],
    )(wids3, x.reshape(b_l + b_s, d), *[al2 for _ in range(n_l)], ws2)

    yl = out[:b_l, :r_l]
    ys = out[b_l:, :r_s]
    return (yl[:, None, :], ys[:, None, :])


# speedup vs baseline: 1.3042x; 1.2297x over previous
import jax
import jax.numpy as jnp
from jax.experimental import pallas as pl

_D = 4096
_NR = 1024
_R_OUT = 64


def _fused_kernel(wids_ref, x_ref, w_ref, out_ref):
    h = pl.program_id(0)
    shift = jnp.where(h == 0, 6, 4)
    rmask = jnp.where(h == 0, 63, 15)

    xb = x_ref[:, 0, :].astype(jnp.bfloat16)
    wb = w_ref[0]
    acc = jnp.dot(xb, wb, preferred_element_type=jnp.float32)
    bm = acc.shape[0]
    wid = wids_ref[0, 0, :]
    lane_e = jax.lax.broadcasted_iota(jnp.int32, (bm, _NR), 1) >> shift
    masked = jnp.where(wid[:, None] == lane_e, acc, 0.0).astype(jnp.bfloat16)
    gi = jax.lax.broadcasted_iota(jnp.int32, (_NR, _R_OUT), 0)
    gj = jax.lax.broadcasted_iota(jnp.int32, (_NR, _R_OUT), 1)
    fold = ((gi & rmask) == gj).astype(jnp.bfloat16)
    out_ref[...] = jnp.dot(masked, fold, preferred_element_type=jnp.float32)


def kernel(x, wids_large, wids_small, lora_A_large, lora_A_small):
    b_l = wids_large.shape[0]
    b_s = wids_small.shape[0]
    n_l, d, r_l = lora_A_large.shape
    n_s, _, r_s = lora_A_small.shape
    bm = 512
    nblk = (b_l + b_s) // bm

    wids3 = jnp.concatenate([wids_large, wids_small]).reshape(nblk, 1, bm)
    wl = lora_A_large.transpose(1, 0, 2).reshape(d, n_l * r_l).astype(jnp.bfloat16)
    ws = lora_A_small.transpose(1, 0, 2).reshape(d, n_s * r_s).astype(jnp.bfloat16)
    w = jnp.stack([wl, ws])

    hb = nblk // 2
    out = pl.pallas_call(
        _fused_kernel,
        grid=(2, hb),
        in_specs=[
            pl.BlockSpec((1, 1, bm), lambda h, i, hb=hb: (h * hb + i, 0, 0)),
            pl.BlockSpec((bm, 1, _D), lambda h, i, hb=hb: (h * hb + i, 0, 0)),
            pl.BlockSpec((1, _D, _NR), lambda h, i: (h, 0, 0)),
        ],
        out_specs=pl.BlockSpec((bm, _R_OUT), lambda h, i, hb=hb: (h * hb + i, 0)),
        out_shape=jax.ShapeDtypeStruct((b_l + b_s, _R_OUT), jnp.float32),
    )(wids3, x, w)

    yl = out[:b_l, :r_l]
    ys = out[b_l:, :r_s]
    return (yl[:, None, :], ys[:, None, :])


# speedup vs baseline: 1.3447x; 1.0310x over previous
import jax
import jax.numpy as jnp
from jax.experimental import pallas as pl

_D = 4096
_NR = 1024
_R_OUT = 64


def _fused_kernel(wids_ref, xa_ref, xb_ref, wa_ref, wb_ref, out_ref):
    h = pl.program_id(0)
    shift = jnp.where(h == 0, 6, 4)
    rmask = jnp.where(h == 0, 63, 15)

    xa = xa_ref[:, 0, :].astype(jnp.bfloat16)
    xc = xb_ref[:, 0, :].astype(jnp.bfloat16)
    acc = (jnp.dot(xa, wa_ref[0], preferred_element_type=jnp.float32)
           + jnp.dot(xc, wb_ref[0], preferred_element_type=jnp.float32))
    bm = acc.shape[0]
    wid = wids_ref[0, 0, :]
    lane_e = jax.lax.broadcasted_iota(jnp.int32, (bm, _NR), 1) >> shift
    masked = jnp.where(wid[:, None] == lane_e, acc, 0.0).astype(jnp.bfloat16)
    gi = jax.lax.broadcasted_iota(jnp.int32, (_NR, _R_OUT), 0)
    gj = jax.lax.broadcasted_iota(jnp.int32, (_NR, _R_OUT), 1)
    fold = ((gi & rmask) == gj).astype(jnp.bfloat16)
    out_ref[...] = jnp.dot(masked, fold, preferred_element_type=jnp.float32)


def kernel(x, wids_large, wids_small, lora_A_large, lora_A_small):
    b_l = wids_large.shape[0]
    b_s = wids_small.shape[0]
    n_l, d, r_l = lora_A_large.shape
    n_s, _, r_s = lora_A_small.shape
    bm = 256
    nblk = (b_l + b_s) // bm

    wids3 = jnp.concatenate([wids_large, wids_small]).reshape(nblk, 1, bm)
    wl = lora_A_large.transpose(1, 0, 2).reshape(d, n_l * r_l).astype(jnp.bfloat16)
    ws = lora_A_small.transpose(1, 0, 2).reshape(d, n_s * r_s).astype(jnp.bfloat16)
    w = jnp.stack([wl, ws])

    hb = nblk // 2
    out = pl.pallas_call(
        _fused_kernel,
        grid=(2, hb),
        in_specs=[
            pl.BlockSpec((1, 1, bm), lambda h, i, hb=hb: (h * hb + i, 0, 0)),
            pl.BlockSpec((bm, 1, _D // 2), lambda h, i, hb=hb: (h * hb + i, 0, 0)),
            pl.BlockSpec((bm, 1, _D // 2), lambda h, i, hb=hb: (h * hb + i, 0, 1)),
            pl.BlockSpec((1, _D // 2, _NR), lambda h, i: (h, 0, 0)),
            pl.BlockSpec((1, _D // 2, _NR), lambda h, i: (h, 1, 0)),
        ],
        out_specs=pl.BlockSpec((bm, _R_OUT), lambda h, i, hb=hb: (h * hb + i, 0)),
        out_shape=jax.ShapeDtypeStruct((b_l + b_s, _R_OUT), jnp.float32),
    )(wids3, x, x, w, w)

    yl = out[:b_l, :r_l]
    ys = out[b_l:, :r_s]
    return (yl[:, None, :], ys[:, None, :])


# speedup vs baseline: 1.3505x; 1.0043x over previous
import jax
import jax.numpy as jnp
from jax.experimental import pallas as pl

_D = 4096
_NR = 1024
_R_OUT = 64


def _fused_kernel(wids_ref, x_ref, w_ref, out_ref):
    h = pl.program_id(0)
    shift = jnp.where(h == 0, 6, 4)
    rmask = jnp.where(h == 0, 63, 15)

    xb = x_ref[:, 0, :].astype(jnp.bfloat16)
    wb = w_ref[0]
    acc = jnp.dot(xb, wb, preferred_element_type=jnp.float32)
    bm = acc.shape[0]
    wid = wids_ref[0, 0, :]
    lane_e = jax.lax.broadcasted_iota(jnp.int32, (bm, _NR), 1) >> shift
    masked = jnp.where(wid[:, None] == lane_e, acc, 0.0).astype(jnp.bfloat16)
    gi = jax.lax.broadcasted_iota(jnp.int32, (_NR, _R_OUT), 0)
    gj = jax.lax.broadcasted_iota(jnp.int32, (_NR, _R_OUT), 1)
    fold = ((gi & rmask) == gj).astype(jnp.bfloat16)
    out_ref[...] = jnp.dot(masked, fold, preferred_element_type=jnp.float32)


def kernel(x, wids_large, wids_small, lora_A_large, lora_A_small):
    b_l = wids_large.shape[0]
    b_s = wids_small.shape[0]
    n_l, d, r_l = lora_A_large.shape
    n_s, _, r_s = lora_A_small.shape
    bm = 256
    nblk = (b_l + b_s) // bm

    wids3 = jnp.concatenate([wids_large, wids_small]).reshape(nblk, 1, bm)
    wl = lora_A_large.transpose(1, 0, 2).reshape(d, n_l * r_l).astype(jnp.bfloat16)
    ws = lora_A_small.transpose(1, 0, 2).reshape(d, n_s * r_s).astype(jnp.bfloat16)
    w = jnp.stack([wl, ws])

    hb = nblk // 2
    out = pl.pallas_call(
        _fused_kernel,
        grid=(2, hb),
        in_specs=[
            pl.BlockSpec((1, 1, bm), lambda h, i, hb=hb: (h * hb + i, 0, 0)),
            pl.BlockSpec((bm, 1, _D), lambda h, i, hb=hb: (h * hb + i, 0, 0)),
            pl.BlockSpec((1, _D, _NR), lambda h, i: (h, 0, 0)),
        ],
        out_specs=pl.BlockSpec((bm, _R_OUT), lambda h, i, hb=hb: (h * hb + i, 0)),
        out_shape=jax.ShapeDtypeStruct((b_l + b_s, _R_OUT), jnp.float32),
    )(wids3, x, w)

    yl = out[:b_l, :r_l]
    ys = out[b_l:, :r_s]
    return (yl[:, None, :], ys[:, None, :])
